# Initial kernel scaffold; baseline (speedup 1.0000x reference)
import jax
import jax.numpy as jnp
from jax.experimental import pallas as pl


def kernel(s_f, s_g, t_g, idx):
    raise NotImplementedError("write your pallas kernel here")


# speedup vs baseline: 5.1987x; 5.1987x over previous
import jax
import jax.numpy as jnp
from jax.experimental import pallas as pl
from jax.experimental.pallas import tpu as pltpu

N = 2048
D = 64
R = 512
G = N // R
TK = 10
TH = 5
_DNT = (((1,), (1,)), ((), ()))
_DN = (((1,), (0,)), ((), ()))
F32 = jnp.float32
BF16 = jnp.bfloat16


def _pc(**kw):
    return pl.pallas_call(**kw)


def _split(x):
    hi = x.astype(BF16)
    lo = (x - hi.astype(F32)).astype(BF16)
    return hi, lo


def _dot3_t(a, b):
    ahi, alo = _split(a)
    bhi, blo = _split(b)
    out = jax.lax.dot_general(ahi, bhi, _DNT, preferred_element_type=F32)
    out = out + jax.lax.dot_general(ahi, blo, _DNT, preferred_element_type=F32)
    out = out + jax.lax.dot_general(alo, bhi, _DNT, preferred_element_type=F32)
    return out


def _sq_row(x):
    xt = jnp.transpose(x)
    return jnp.sum(xt * xt, axis=0, keepdims=True)


def _sq_col(x):
    return jnp.sum(x * x, axis=1, keepdims=True)


def _dist_tile(xb, xf):
    d2 = _sq_col(xb) + _sq_row(xf) - 2.0 * _dot3_t(xb, xf)
    return jnp.sqrt(jnp.maximum(d2, 1e-12))


def _norm_rows(x):
    n = jnp.sqrt(jnp.sum(x * x, axis=1, keepdims=True))
    return x / jnp.maximum(n, 1e-12)


def _p1(tb, tf, ic, ir, wnn_ref, h5_ref):
    tnb = _norm_rows(tb[...])
    tnf = _norm_rows(tf[...])
    d2 = jnp.maximum(_sq_col(tnb) + _sq_row(tnf) - 2.0 * _dot3_t(tnb, tnf),
                     1e-12)
    wp = jnp.exp(-d2)
    same = ic[...] == ir[...]
    wpc = jnp.where(same, 1.0, wp)
    iota = jax.lax.broadcasted_iota(jnp.int32, (R, N), 1)
    acc = jnp.zeros((R, N), jnp.bool_)
    h5 = acc
    for it in range(TK):
        m = jnp.max(wpc, axis=1, keepdims=True)
        cand = jnp.where(wpc == m, iota, N)
        j = jnp.min(cand, axis=1, keepdims=True)
        onehot = iota == j
        acc = jnp.logical_or(acc, onehot)
        if it == TH - 1:
            h5 = acc
        wpc = jnp.where(onehot, -1.0, wpc)
    wnn_ref[...] = acc.astype(BF16)
    h5_ref[...] = h5.astype(BF16)


def _p2(wr, wc, v_ref, rc_ref):
    v = wr[...].astype(F32) * jnp.transpose(wc[...].astype(F32))
    v_ref[...] = v.astype(BF16)
    rc_ref[...] = jnp.sum(v, axis=1, keepdims=True)


def _p3(vb, vf, rc, wct_ref):
    vv = jax.lax.dot_general(vb[...], vf[...], _DNT,
                             preferred_element_type=F32)
    wct = vb[...].astype(F32) * vv / jnp.maximum(rc[...], 1.0)
    wct_ref[...] = wct.astype(BF16)


def _p4(h5, wct, fb, ff, gb, gf,
        what_ref, rsf_ref, lsea_ref, rsg_ref, lseb_ref):
    what = jax.lax.dot_general(h5[...], wct[...], _DN,
                               preferred_element_type=F32)
    what_ref[...] = what * (1.0 / TH)
    sf = _dist_tile(fb[...], ff[...])
    rsf = jnp.sum(sf, axis=1, keepdims=True)
    rsf_ref[...] = rsf
    sfn = sf * (N / rsf)
    lsea_ref[...] = jnp.log(jnp.sum(jnp.exp(-sfn), axis=1, keepdims=True))
    sg = _dist_tile(gb[...], gf[...])
    rsg = jnp.sum(sg, axis=1, keepdims=True)
    rsg_ref[...] = rsg
    sgn = sg * (N / rsg)
    lseb_ref[...] = jnp.log(jnp.sum(jnp.exp(-sgn), axis=1, keepdims=True))


def _p5(tb, tf, fb, ff, gb, gf, wr, wc, rsf, rsg, lsea, lseb,
        rcf_ref, rcg_ref, kl_ref):
    i = pl.program_id(0)
    tnb = _norm_rows(tb[...])
    tnf = _norm_rows(tf[...])
    d2 = jnp.maximum(_sq_col(tnb) + _sq_row(tnf) - 2.0 * _dot3_t(tnb, tnf),
                     1e-12)
    wp = jnp.exp(-d2)
    w = 0.5 * (wp + 0.5 * (wr[...] + jnp.transpose(wc[...])))
    sfn = _dist_tile(fb[...], ff[...]) * (N / rsf[...])
    sgn = _dist_tile(gb[...], gf[...]) * (N / rsg[...])
    rows = i * R + jax.lax.broadcasted_iota(jnp.int32, (R, N), 0)
    cols = jax.lax.broadcasted_iota(jnp.int32, (R, N), 1)
    offd = (rows != cols).astype(F32)
    hf = jnp.maximum(1.0 - sfn, 0.0)
    hg = jnp.maximum(1.0 - sgn, 0.0)
    rcf = jnp.sum((sfn * sfn * w + hf * hf * (1.0 - w)) * offd,
                  keepdims=True)
    rcg = jnp.sum((sgn * sgn * w + hg * hg * (1.0 - w)) * offd,
                  keepdims=True)
    p = jnp.exp(-sgn - lseb[...])
    kl = (jnp.sum(p * (sfn - sgn), keepdims=True)
          + jnp.sum(lsea[...] - lseb[...], keepdims=True))

    @pl.when(i == 0)
    def _init():
        rcf_ref[...] = jnp.zeros((1, 1), F32)
        rcg_ref[...] = jnp.zeros((1, 1), F32)
        kl_ref[...] = jnp.zeros((1, 1), F32)

    rcf_ref[...] += rcf
    rcg_ref[...] += rcg
    kl_ref[...] += kl


def _blk(shape, im):
    return pl.BlockSpec(shape, im)


def _i0(i):
    return (i, 0)


def _00(i):
    return (0, 0)


def _0i(i):
    return (0, i)


_VMEM = pltpu.CompilerParams(vmem_limit_bytes=56 * 2**20)


def kernel(s_f, s_g, t_g, idx):
    idx = idx.astype(jnp.int32)
    idxc = idx.reshape(N, 1)
    idxr = idx.reshape(1, N)

    eblk = _blk((R, D), _i0)
    efull = _blk((N, D), _00)
    col = _blk((R, 1), _i0)
    rowstrip = _blk((R, N), _i0)
    colstrip = _blk((N, R), _0i)
    fullmat = _blk((N, N), _00)

    wnn, h5 = _pc(
        kernel=_p1,
        grid=(G,),
        in_specs=[eblk, efull, col, _blk((1, N), _00)],
        out_specs=[rowstrip, rowstrip],
        out_shape=[jax.ShapeDtypeStruct((N, N), BF16)] * 2,
        compiler_params=_VMEM,
    )(t_g, t_g, idxc, idxr)

    v, rc = _pc(
        kernel=_p2,
        grid=(G,),
        in_specs=[rowstrip, colstrip],
        out_specs=[rowstrip, col],
        out_shape=[jax.ShapeDtypeStruct((N, N), BF16),
                   jax.ShapeDtypeStruct((N, 1), F32)],
        compiler_params=_VMEM,
    )(wnn, wnn)

    wct, = _pc(
        kernel=_p3,
        grid=(G,),
        in_specs=[rowstrip, fullmat, col],
        out_specs=[rowstrip],
        out_shape=[jax.ShapeDtypeStruct((N, N), BF16)],
        compiler_params=_VMEM,
    )(v, v, rc)

    what, rsf, lsea, rsg, lseb = _pc(
        kernel=_p4,
        grid=(G,),
        in_specs=[rowstrip, fullmat, eblk, efull, eblk, efull],
        out_specs=[rowstrip, col, col, col, col],
        out_shape=[jax.ShapeDtypeStruct((N, N), F32)] +
                  [jax.ShapeDtypeStruct((N, 1), F32)] * 4,
        compiler_params=_VMEM,
    )(h5, wct, s_f, s_f, s_g, s_g)

    one = _blk((1, 1), _00)
    rcf, rcg, kl = _pc(
        kernel=_p5,
        grid=(G,),
        in_specs=[eblk, efull, eblk, efull, eblk, efull,
                  rowstrip, colstrip, col, col, col, col],
        out_specs=[one, one, one],
        out_shape=[jax.ShapeDtypeStruct((1, 1), F32)] * 3,
        compiler_params=_VMEM,
    )(t_g, t_g, s_f, s_f, s_g, s_g, what, what, rsf, rsg, lsea, lseb)

    scale = 1.0 / (N * (N - 1))
    loss_rc = 0.5 * (rcf[0, 0] + rcg[0, 0]) * scale
    loss_kl = kl[0, 0] / N
    return (loss_rc, loss_kl, loss_rc + loss_kl)


# speedup vs baseline: 5.2500x; 1.0099x over previous
import jax
import jax.numpy as jnp
from jax.experimental import pallas as pl
from jax.experimental.pallas import tpu as pltpu

N = 2048
D = 64
R1 = 512
R2 = 256
TK = 10
TH = 5
_DNT = (((1,), (1,)), ((), ()))
_DN = (((1,), (0,)), ((), ()))
F32 = jnp.float32
BF16 = jnp.bfloat16


def _pc(**kw):
    return pl.pallas_call(**kw)


def _split(x):
    hi = x.astype(BF16)
    lo = (x - hi.astype(F32)).astype(BF16)
    return hi, lo


def _dot3_t(a, b):
    ahi, alo = _split(a)
    bhi, blo = _split(b)
    out = jax.lax.dot_general(ahi, bhi, _DNT, preferred_element_type=F32)
    out = out + jax.lax.dot_general(ahi, blo, _DNT, preferred_element_type=F32)
    out = out + jax.lax.dot_general(alo, bhi, _DNT, preferred_element_type=F32)
    return out


def _sq_row(x):
    xt = jnp.transpose(x)
    return jnp.sum(xt * xt, axis=0, keepdims=True)


def _sq_col(x):
    return jnp.sum(x * x, axis=1, keepdims=True)


def _dist_tile(xb, xf):
    d2 = _sq_col(xb) + _sq_row(xf) - 2.0 * _dot3_t(xb, xf)
    return jnp.sqrt(jnp.maximum(d2, 1e-12))


def _norm_rows(x):
    n = jnp.sqrt(jnp.sum(x * x, axis=1, keepdims=True))
    return x / jnp.maximum(n, 1e-12)


def _s1(fb, ff, gb, gf, rsf_ref, lsea_ref, rsg_ref, lseb_ref):
    sf = _dist_tile(fb[...], ff[...])
    rsf = jnp.sum(sf, axis=1, keepdims=True)
    rsf_ref[...] = rsf
    lsea_ref[...] = jnp.log(jnp.sum(jnp.exp(-sf * (N / rsf)),
                                    axis=1, keepdims=True))
    sg = _dist_tile(gb[...], gf[...])
    rsg = jnp.sum(sg, axis=1, keepdims=True)
    rsg_ref[...] = rsg
    lseb_ref[...] = jnp.log(jnp.sum(jnp.exp(-sg * (N / rsg)),
                                    axis=1, keepdims=True))


def _s2(tb, tf, ic, ir, wnn_ref, wnnt_ref, h5_ref):
    tnb = _norm_rows(tb[...])
    tnf = _norm_rows(tf[...])
    d2 = jnp.maximum(_sq_col(tnb) + _sq_row(tnf) - 2.0 * _dot3_t(tnb, tnf),
                     1e-12)
    wp = jnp.exp(-d2)
    same = ic[...] == ir[...]
    wpc = jnp.where(same, 1.0, wp)
    iota = jax.lax.broadcasted_iota(jnp.int32, (R1, N), 1)
    acc = jnp.zeros((R1, N), jnp.bool_)
    h5 = acc
    for it in range(TK):
        m = jnp.max(wpc, axis=1, keepdims=True)
        cand = jnp.where(wpc == m, iota, N)
        j = jnp.min(cand, axis=1, keepdims=True)
        onehot = iota == j
        acc = jnp.logical_or(acc, onehot)
        if it == TH - 1:
            h5 = acc
        wpc = jnp.where(onehot, -1.0, wpc)
    accf = acc.astype(F32)
    wnn_ref[...] = accf.astype(BF16)
    wnnt_ref[...] = jnp.transpose(accf).astype(BF16)
    h5_ref[...] = h5.astype(BF16)


def _s3(wb, wtb, wf, wtf, wct_ref):
    vb = wb[...] * wtb[...]
    vf = wf[...] * wtf[...]
    vv = jax.lax.dot_general(vb, vf, _DNT, preferred_element_type=F32)
    rc = jnp.sum(vb.astype(F32), axis=1, keepdims=True)
    wct = vb.astype(F32) * vv / jnp.maximum(rc, 1.0)
    wct_ref[...] = wct.astype(BF16)


def _s4(h5, wct, tb, tf, fb, ff, gb, gf,
        rsf, rsg, rsft, rsgt, lsea, lseb, rcf_ref, rcg_ref, kl_ref):
    i = pl.program_id(0)
    what = jax.lax.dot_general(h5[...], wct[...], _DN,
                               preferred_element_type=F32) * (1.0 / TH)
    tnb = _norm_rows(tb[...])
    tnf = _norm_rows(tf[...])
    d2 = jnp.maximum(_sq_col(tnb) + _sq_row(tnf) - 2.0 * _dot3_t(tnb, tnf),
                     1e-12)
    wp = jnp.exp(-d2)
    rows = i * R2 + jax.lax.broadcasted_iota(jnp.int32, (R2, N), 0)
    cols = jax.lax.broadcasted_iota(jnp.int32, (R2, N), 1)
    offd = (rows != cols).astype(F32)
    wpo = wp * offd
    whato = what * offd

    def terms(sb, sfull, rs, rst):
        s = _dist_tile(sb, sfull)
        sn = s * (N / rs)
        snt = s * (N / rst)
        h = jnp.maximum(1.0 - sn, 0.0)
        ht = jnp.maximum(1.0 - snt, 0.0)
        f = sn * sn - h * h
        ft = snt * snt - ht * ht
        g2 = 0.5 * (f + ft)
        push = jnp.sum(h * h * offd, keepdims=True)
        wpterm = jnp.sum(wpo * f, keepdims=True)
        wcterm = jnp.sum(whato * g2, keepdims=True)
        return push + 0.5 * (wpterm + wcterm), sn

    lf, sfn = terms(fb[...], ff[...], rsf[...], rsft[...])
    lg, sgn = terms(gb[...], gf[...], rsg[...], rsgt[...])
    p = jnp.exp(-sgn - lseb[...])
    kl = (jnp.sum(p * (sfn - sgn), keepdims=True)
          + jnp.sum(lsea[...] - lseb[...], keepdims=True))

    @pl.when(i == 0)
    def _init():
        rcf_ref[...] = jnp.zeros((1, 1), F32)
        rcg_ref[...] = jnp.zeros((1, 1), F32)
        kl_ref[...] = jnp.zeros((1, 1), F32)

    rcf_ref[...] += lf
    rcg_ref[...] += lg
    kl_ref[...] += kl


def _blk(shape, im):
    return pl.BlockSpec(shape, im)


def _i0(i):
    return (i, 0)


def _00(i):
    return (0, 0)


def _0i(i):
    return (0, i)


_VMEM = pltpu.CompilerParams(vmem_limit_bytes=56 * 2**20)


def kernel(s_f, s_g, t_g, idx):
    idx = idx.astype(jnp.int32)
    idxc = idx.reshape(N, 1)
    idxr = idx.reshape(1, N)

    e1 = _blk((R1, D), _i0)
    ef = _blk((N, D), _00)
    c1 = _blk((R1, 1), _i0)
    r1 = _blk((R1, N), _i0)
    fullmat = _blk((N, N), _00)

    rsf, lsea, rsg, lseb = _pc(
        kernel=_s1,
        grid=(N // R1,),
        in_specs=[e1, ef, e1, ef],
        out_specs=[c1, c1, c1, c1],
        out_shape=[jax.ShapeDtypeStruct((N, 1), F32)] * 4,
        compiler_params=_VMEM,
    )(s_f, s_f, s_g, s_g)

    wnn, wnnt, h5 = _pc(
        kernel=_s2,
        grid=(N // R1,),
        in_specs=[e1, ef, c1, _blk((1, N), _00)],
        out_specs=[r1, _blk((N, R1), _0i), r1],
        out_shape=[jax.ShapeDtypeStruct((N, N), BF16)] * 3,
        compiler_params=_VMEM,
    )(t_g, t_g, idxc, idxr)

    wct, = _pc(
        kernel=_s3,
        grid=(N // R1,),
        in_specs=[r1, r1, fullmat, fullmat],
        out_specs=[r1],
        out_shape=[jax.ShapeDtypeStruct((N, N), BF16)],
        compiler_params=_VMEM,
    )(wnn, wnnt, wnn, wnnt)

    e2 = _blk((R2, D), _i0)
    c2 = _blk((R2, 1), _i0)
    r2 = _blk((R2, N), _i0)
    one = _blk((1, 1), _00)
    rsft = rsf.reshape(1, N)
    rsgt = rsg.reshape(1, N)
    rcf, rcg, kl = _pc(
        kernel=_s4,
        grid=(N // R2,),
        in_specs=[r2, fullmat, e2, ef, e2, ef, e2, ef,
                  c2, c2, _blk((1, N), _00), _blk((1, N), _00), c2, c2],
        out_specs=[one, one, one],
        out_shape=[jax.ShapeDtypeStruct((1, 1), F32)] * 3,
        compiler_params=_VMEM,
    )(h5, wct, t_g, t_g, s_f, s_f, s_g, s_g,
      rsf, rsg, rsft, rsgt, lsea, lseb)

    scale = 1.0 / (N * (N - 1))
    loss_rc = 0.5 * (rcf[0, 0] + rcg[0, 0]) * scale
    loss_kl = kl[0, 0] / N
    return (loss_rc, loss_kl, loss_rc + loss_kl)


# speedup vs baseline: 6.1857x; 1.1782x over previous
import jax
import jax.numpy as jnp
from jax.experimental import pallas as pl
from jax.experimental.pallas import tpu as pltpu

N = 2048
D = 64
R1 = 512
R2 = 256
TK = 10
TH = 5
_DNT = (((1,), (1,)), ((), ()))
_DN = (((1,), (0,)), ((), ()))
F32 = jnp.float32
BF16 = jnp.bfloat16


def _pc(**kw):
    return pl.pallas_call(**kw)


def _split(x):
    hi = x.astype(BF16)
    lo = (x - hi.astype(F32)).astype(BF16)
    return hi, lo


def _dot3_t(a, b):
    ahi, alo = _split(a)
    bhi, blo = _split(b)
    out = jax.lax.dot_general(ahi, bhi, _DNT, preferred_element_type=F32)
    out = out + jax.lax.dot_general(ahi, blo, _DNT, preferred_element_type=F32)
    out = out + jax.lax.dot_general(alo, bhi, _DNT, preferred_element_type=F32)
    return out


def _sq_row(x):
    xt = jnp.transpose(x)
    return jnp.sum(xt * xt, axis=0, keepdims=True)


def _sq_col(x):
    return jnp.sum(x * x, axis=1, keepdims=True)


def _dist_tile(xb, xf):
    d2 = _sq_col(xb) + _sq_row(xf) - 2.0 * _dot3_t(xb, xf)
    return jnp.sqrt(jnp.maximum(d2, 1e-12))


def _norm_rows(x):
    n = jnp.sqrt(jnp.sum(x * x, axis=1, keepdims=True))
    return x / jnp.maximum(n, 1e-12)


def _s2(tb, tf, ic, ir, wnn_ref, wnnt_ref, h5_ref):
    tnb = _norm_rows(tb[...])
    tnf = _norm_rows(tf[...])
    d2 = jnp.maximum(_sq_col(tnb) + _sq_row(tnf) - 2.0 * _dot3_t(tnb, tnf),
                     1e-12)
    wp = jnp.exp(-d2)
    same = ic[...] == ir[...]
    wpc = jnp.where(same, 1.0, wp)
    iota = jax.lax.broadcasted_iota(jnp.int32, (R1, N), 1)
    acc = jnp.zeros((R1, N), jnp.bool_)
    h5 = acc
    for it in range(TK):
        m = jnp.max(wpc, axis=1, keepdims=True)
        cand = jnp.where(wpc == m, iota, N)
        j = jnp.min(cand, axis=1, keepdims=True)
        onehot = iota == j
        acc = jnp.logical_or(acc, onehot)
        if it == TH - 1:
            h5 = acc
        wpc = jnp.where(onehot, -1.0, wpc)
    accf = acc.astype(F32)
    wnn_ref[...] = accf.astype(BF16)
    wnnt_ref[...] = jnp.transpose(accf).astype(BF16)
    h5_ref[...] = h5.astype(BF16)


def _s3(wb, wtb, wf, wtf, wct_ref):
    vb = wb[...] * wtb[...]
    vf = wf[...] * wtf[...]
    vv = jax.lax.dot_general(vb, vf, _DNT, preferred_element_type=F32)
    rc = jnp.sum(vb.astype(F32), axis=1, keepdims=True)
    wct = vb.astype(F32) * vv / jnp.maximum(rc, 1.0)
    wct_ref[...] = wct.astype(BF16)


def _sa(h5, wct, what_ref, whatt_ref):
    w = jax.lax.dot_general(h5[...], wct[...], _DN,
                            preferred_element_type=F32) * (1.0 / TH)
    what_ref[...] = w.astype(BF16)
    whatt_ref[...] = jnp.transpose(w).astype(BF16)


def _sb(wr, wtr, tb, tf, fb, ff, gb, gf, rcf_ref, rcg_ref, kl_ref):
    i = pl.program_id(0)
    tnb = _norm_rows(tb[...])
    tnf = _norm_rows(tf[...])
    d2 = jnp.maximum(_sq_col(tnb) + _sq_row(tnf) - 2.0 * _dot3_t(tnb, tnf),
                     1e-12)
    wp = jnp.exp(-d2)
    wc = 0.5 * (wr[...].astype(F32) + wtr[...].astype(F32))
    w = 0.5 * (wp + wc)
    rows = i * R2 + jax.lax.broadcasted_iota(jnp.int32, (R2, N), 0)
    cols = jax.lax.broadcasted_iota(jnp.int32, (R2, N), 1)
    offd = (rows != cols).astype(F32)

    def terms(sb_, sfull):
        s = _dist_tile(sb_, sfull)
        rs = jnp.sum(s, axis=1, keepdims=True)
        sn = s * (N / rs)
        lse = jnp.log(jnp.sum(jnp.exp(-sn), axis=1, keepdims=True))
        h = jnp.maximum(1.0 - sn, 0.0)
        loss = jnp.sum((sn * sn * w + h * h * (1.0 - w)) * offd,
                       keepdims=True)
        return loss, sn, lse

    lf, sfn, lsea = terms(fb[...], ff[...])
    lg, sgn, lseb = terms(gb[...], gf[...])
    p = jnp.exp(-sgn - lseb)
    kl = (jnp.sum(p * (sfn - sgn), keepdims=True)
          + jnp.sum(lsea - lseb, keepdims=True))

    @pl.when(i == 0)
    def _init():
        rcf_ref[...] = jnp.zeros((1, 1), F32)
        rcg_ref[...] = jnp.zeros((1, 1), F32)
        kl_ref[...] = jnp.zeros((1, 1), F32)

    rcf_ref[...] += lf
    rcg_ref[...] += lg
    kl_ref[...] += kl


def _blk(shape, im):
    return pl.BlockSpec(shape, im)


def _i0(i):
    return (i, 0)


def _00(i):
    return (0, 0)


def _0i(i):
    return (0, i)


_VMEM = pltpu.CompilerParams(vmem_limit_bytes=56 * 2**20)


def kernel(s_f, s_g, t_g, idx):
    idx = idx.astype(jnp.int32)
    idxc = idx.reshape(N, 1)
    idxr = idx.reshape(1, N)

    e1 = _blk((R1, D), _i0)
    ef = _blk((N, D), _00)
    c1 = _blk((R1, 1), _i0)
    r1 = _blk((R1, N), _i0)
    fullmat = _blk((N, N), _00)

    wnn, wnnt, h5 = _pc(
        kernel=_s2,
        grid=(N // R1,),
        in_specs=[e1, ef, c1, _blk((1, N), _00)],
        out_specs=[r1, _blk((N, R1), _0i), r1],
        out_shape=[jax.ShapeDtypeStruct((N, N), BF16)] * 3,
        compiler_params=_VMEM,
    )(t_g, t_g, idxc, idxr)

    wct, = _pc(
        kernel=_s3,
        grid=(N // R1,),
        in_specs=[r1, r1, fullmat, fullmat],
        out_specs=[r1],
        out_shape=[jax.ShapeDtypeStruct((N, N), BF16)],
        compiler_params=_VMEM,
    )(wnn, wnnt, wnn, wnnt)

    what, whatt = _pc(
        kernel=_sa,
        grid=(N // R1,),
        in_specs=[r1, fullmat],
        out_specs=[r1, _blk((N, R1), _0i)],
        out_shape=[jax.ShapeDtypeStruct((N, N), BF16)] * 2,
        compiler_params=_VMEM,
    )(h5, wct)

    e2 = _blk((R2, D), _i0)
    r2 = _blk((R2, N), _i0)
    one = _blk((1, 1), _00)
    rcf, rcg, kl = _pc(
        kernel=_sb,
        grid=(N // R2,),
        in_specs=[r2, r2, e2, ef, e2, ef, e2, ef],
        out_specs=[one, one, one],
        out_shape=[jax.ShapeDtypeStruct((1, 1), F32)] * 3,
        compiler_params=_VMEM,
    )(what, whatt, t_g, t_g, s_f, s_f, s_g, s_g)

    scale = 1.0 / (N * (N - 1))
    loss_rc = 0.5 * (rcf[0, 0] + rcg[0, 0]) * scale
    loss_kl = kl[0, 0] / N
    return (loss_rc, loss_kl, loss_rc + loss_kl)


# speedup vs baseline: 6.6097x; 1.0685x over previous
import jax
import jax.numpy as jnp
from jax.experimental import pallas as pl
from jax.experimental.pallas import tpu as pltpu

N = 2048
D = 64
R1 = 512
R2 = 256
TK = 10
TH = 5
_DNT = (((1,), (1,)), ((), ()))
_DN = (((1,), (0,)), ((), ()))
F32 = jnp.float32
BF16 = jnp.bfloat16


def _pc(**kw):
    return pl.pallas_call(**kw)


def _split(x):
    hi = x.astype(BF16)
    lo = (x - hi.astype(F32)).astype(BF16)
    return hi, lo


def _dot3_t(a, b):
    ahi, alo = _split(a)
    bhi, blo = _split(b)
    out = jax.lax.dot_general(ahi, bhi, _DNT, preferred_element_type=F32)
    out = out + jax.lax.dot_general(ahi, blo, _DNT, preferred_element_type=F32)
    out = out + jax.lax.dot_general(alo, bhi, _DNT, preferred_element_type=F32)
    return out


def _sq_row(x):
    xt = jnp.transpose(x)
    return jnp.sum(xt * xt, axis=0, keepdims=True)


def _sq_col(x):
    return jnp.sum(x * x, axis=1, keepdims=True)


def _dist_tile(xb, xf):
    d2 = _sq_col(xb) + _sq_row(xf) - 2.0 * _dot3_t(xb, xf)
    return jnp.sqrt(jnp.maximum(d2, 1e-12))


def _norm_rows(x):
    n = jnp.sqrt(jnp.sum(x * x, axis=1, keepdims=True))
    return x / jnp.maximum(n, 1e-12)


def _s2(tb, tf, ic, ir, wnn_ref, wnnt_ref, h5_ref):
    tnb = _norm_rows(tb[...])
    tnf = _norm_rows(tf[...])
    d2 = jnp.maximum(_sq_col(tnb) + _sq_row(tnf) - 2.0 * _dot3_t(tnb, tnf),
                     1e-12)
    wp = jnp.exp(-d2)
    same = ic[...] == ir[...]
    wpc = jnp.where(same, 1.0, wp)
    iota = jax.lax.broadcasted_iota(jnp.int32, (R1, N), 1)
    h5 = jnp.zeros((R1, N), jnp.bool_)
    for it in range(TK):
        m = jnp.max(wpc, axis=1, keepdims=True)
        j = jnp.min(jnp.where(wpc == m, iota, N), axis=1, keepdims=True)
        wpc = jnp.where(iota == j, -1.0, wpc)
        if it == TH - 1:
            h5 = wpc < 0.0
    accf = (wpc < 0.0).astype(F32)
    wnn_ref[...] = accf.astype(BF16)
    wnnt_ref[...] = jnp.transpose(accf).astype(BF16)
    h5_ref[...] = h5.astype(BF16)


def _s3(wb, wtb, wf, wtf, wct_ref):
    vb = wb[...] * wtb[...]
    vf = wf[...] * wtf[...]
    vv = jax.lax.dot_general(vb, vf, _DNT, preferred_element_type=F32)
    rc = jnp.sum(vb.astype(F32), axis=1, keepdims=True)
    wct = vb.astype(F32) * vv / jnp.maximum(rc, 1.0)
    wct_ref[...] = wct.astype(BF16)


def _sa(h5, wct, what_ref, whatt_ref):
    w = jax.lax.dot_general(h5[...], wct[...], _DN,
                            preferred_element_type=F32) * (1.0 / TH)
    what_ref[...] = w.astype(BF16)
    whatt_ref[...] = jnp.transpose(w).astype(BF16)


def _sb(wr, wtr, tb, tf, fb, ff, gb, gf, rcf_ref, rcg_ref, kl_ref):
    i = pl.program_id(0)
    tnb = _norm_rows(tb[...])
    tnf = _norm_rows(tf[...])
    d2 = jnp.maximum(_sq_col(tnb) + _sq_row(tnf) - 2.0 * _dot3_t(tnb, tnf),
                     1e-12)
    wp = jnp.exp(-d2)
    wc = 0.5 * (wr[...].astype(F32) + wtr[...].astype(F32))
    w = 0.5 * (wp + wc)
    rows = i * R2 + jax.lax.broadcasted_iota(jnp.int32, (R2, N), 0)
    cols = jax.lax.broadcasted_iota(jnp.int32, (R2, N), 1)
    offd = (rows != cols).astype(F32)

    def terms(sb_, sfull):
        s = _dist_tile(sb_, sfull)
        rs = jnp.sum(s, axis=1, keepdims=True)
        sn = s * (N / rs)
        lse = jnp.log(jnp.sum(jnp.exp(-sn), axis=1, keepdims=True))
        h = jnp.maximum(1.0 - sn, 0.0)
        loss = jnp.sum((sn * sn * w + h * h * (1.0 - w)) * offd,
                       keepdims=True)
        return loss, sn, lse

    lf, sfn, lsea = terms(fb[...], ff[...])
    lg, sgn, lseb = terms(gb[...], gf[...])
    p = jnp.exp(-sgn - lseb)
    kl = (jnp.sum(p * (sfn - sgn), keepdims=True)
          + jnp.sum(lsea - lseb, keepdims=True))

    @pl.when(i == 0)
    def _init():
        rcf_ref[...] = jnp.zeros((1, 1), F32)
        rcg_ref[...] = jnp.zeros((1, 1), F32)
        kl_ref[...] = jnp.zeros((1, 1), F32)

    rcf_ref[...] += lf
    rcg_ref[...] += lg
    kl_ref[...] += kl


def _blk(shape, im):
    return pl.BlockSpec(shape, im)


def _i0(i):
    return (i, 0)


def _00(i):
    return (0, 0)


def _0i(i):
    return (0, i)


_VMEM = pltpu.CompilerParams(vmem_limit_bytes=56 * 2**20)


def kernel(s_f, s_g, t_g, idx):
    idx = idx.astype(jnp.int32)
    idxc = idx.reshape(N, 1)
    idxr = idx.reshape(1, N)

    e1 = _blk((R1, D), _i0)
    ef = _blk((N, D), _00)
    c1 = _blk((R1, 1), _i0)
    r1 = _blk((R1, N), _i0)
    fullmat = _blk((N, N), _00)

    wnn, wnnt, h5 = _pc(
        kernel=_s2,
        grid=(N // R1,),
        in_specs=[e1, ef, c1, _blk((1, N), _00)],
        out_specs=[r1, _blk((N, R1), _0i), r1],
        out_shape=[jax.ShapeDtypeStruct((N, N), BF16)] * 3,
        compiler_params=_VMEM,
    )(t_g, t_g, idxc, idxr)

    wct, = _pc(
        kernel=_s3,
        grid=(N // R1,),
        in_specs=[r1, r1, fullmat, fullmat],
        out_specs=[r1],
        out_shape=[jax.ShapeDtypeStruct((N, N), BF16)],
        compiler_params=_VMEM,
    )(wnn, wnnt, wnn, wnnt)

    what, whatt = _pc(
        kernel=_sa,
        grid=(N // R1,),
        in_specs=[r1, fullmat],
        out_specs=[r1, _blk((N, R1), _0i)],
        out_shape=[jax.ShapeDtypeStruct((N, N), BF16)] * 2,
        compiler_params=_VMEM,
    )(h5, wct)

    e2 = _blk((R2, D), _i0)
    r2 = _blk((R2, N), _i0)
    one = _blk((1, 1), _00)
    rcf, rcg, kl = _pc(
        kernel=_sb,
        grid=(N // R2,),
        in_specs=[r2, r2, e2, ef, e2, ef, e2, ef],
        out_specs=[one, one, one],
        out_shape=[jax.ShapeDtypeStruct((1, 1), F32)] * 3,
        compiler_params=_VMEM,
    )(what, whatt, t_g, t_g, s_f, s_f, s_g, s_g)

    scale = 1.0 / (N * (N - 1))
    loss_rc = 0.5 * (rcf[0, 0] + rcg[0, 0]) * scale
    loss_kl = kl[0, 0] / N
    return (loss_rc, loss_kl, loss_rc + loss_kl)


# speedup vs baseline: 7.3367x; 1.1100x over previous
import jax
import jax.numpy as jnp
from jax.experimental import pallas as pl
from jax.experimental.pallas import tpu as pltpu

N = 2048
D = 64
R1 = 512
R2 = 256
TK = 10
TH = 5
_DNT = (((1,), (1,)), ((), ()))
_DN = (((1,), (0,)), ((), ()))
F32 = jnp.float32
BF16 = jnp.bfloat16


def _split(x):
    hi = x.astype(BF16)
    lo = (x - hi.astype(F32)).astype(BF16)
    return hi, lo


def _dot3_t(a, b):
    ahi, alo = _split(a)
    bhi, blo = _split(b)
    out = jax.lax.dot_general(ahi, bhi, _DNT, preferred_element_type=F32)
    out = out + jax.lax.dot_general(ahi, blo, _DNT, preferred_element_type=F32)
    out = out + jax.lax.dot_general(alo, bhi, _DNT, preferred_element_type=F32)
    return out


def _sq_row(x):
    xt = jnp.transpose(x)
    return jnp.sum(xt * xt, axis=0, keepdims=True)


def _sq_col(x):
    return jnp.sum(x * x, axis=1, keepdims=True)


def _dist_tile(xb, xf):
    d2 = _sq_col(xb) + _sq_row(xf) - 2.0 * _dot3_t(xb, xf)
    return jnp.sqrt(jnp.maximum(d2, 1e-12))


def _norm_rows(x):
    n = jnp.sqrt(jnp.sum(x * x, axis=1, keepdims=True))
    return x / jnp.maximum(n, 1e-12)


def _mega(t_ref, ic_ref, ir_ref, f_ref, g_ref,
          rcf_ref, rcg_ref, kl_ref,
          a_ref, b_ref, h_ref, d_ref, e_ref):
    i = pl.program_id(0)

    @pl.when(i < 4)
    def _s2():
        blk = i * R1
        tnb = _norm_rows(t_ref[pl.ds(blk, R1), :])
        tnf = _norm_rows(t_ref[...])
        d2 = jnp.maximum(
            _sq_col(tnb) + _sq_row(tnf) - 2.0 * _dot3_t(tnb, tnf), 1e-12)
        wp = jnp.exp(-d2)
        same = ic_ref[pl.ds(blk, R1), :] == ir_ref[...]
        wpc = jnp.where(same, 1.0, wp)
        iota = jax.lax.broadcasted_iota(jnp.int32, (R1, N), 1)
        h5 = jnp.zeros((R1, N), jnp.bool_)
        for it in range(TK):
            m = jnp.max(wpc, axis=1, keepdims=True)
            j = jnp.min(jnp.where(wpc == m, iota, N), axis=1, keepdims=True)
            wpc = jnp.where(iota == j, -1.0, wpc)
            if it == TH - 1:
                h5 = wpc < 0.0
        accf = (wpc < 0.0).astype(F32)
        a_ref[pl.ds(blk, R1), :] = accf.astype(BF16)
        b_ref[:, pl.ds(blk, R1)] = jnp.transpose(accf).astype(BF16)
        h_ref[pl.ds(blk, R1), :] = h5.astype(BF16)

    @pl.when(jnp.logical_and(i >= 4, i < 8))
    def _s3a():
        blk = (i - 4) * R1
        a_ref[pl.ds(blk, R1), :] = (a_ref[pl.ds(blk, R1), :]
                                    * b_ref[pl.ds(blk, R1), :])

    @pl.when(jnp.logical_and(i >= 8, i < 12))
    def _s3b():
        blk = (i - 8) * R1
        vb = a_ref[pl.ds(blk, R1), :]
        vv = jax.lax.dot_general(vb, a_ref[...], _DNT,
                                 preferred_element_type=F32)
        rc = jnp.sum(vb.astype(F32), axis=1, keepdims=True)
        wct = vb.astype(F32) * vv / jnp.maximum(rc, 1.0)
        b_ref[pl.ds(blk, R1), :] = wct.astype(BF16)

    @pl.when(jnp.logical_and(i >= 12, i < 16))
    def _sa():
        blk = (i - 12) * R1
        w = jax.lax.dot_general(h_ref[pl.ds(blk, R1), :], b_ref[...], _DN,
                                preferred_element_type=F32) * (1.0 / TH)
        d_ref[pl.ds(blk, R1), :] = w.astype(BF16)
        e_ref[:, pl.ds(blk, R1)] = jnp.transpose(w).astype(BF16)

    @pl.when(i >= 16)
    def _sb():
        k = i - 16
        blk = k * R2
        tnb = _norm_rows(t_ref[pl.ds(blk, R2), :])
        tnf = _norm_rows(t_ref[...])
        d2 = jnp.maximum(
            _sq_col(tnb) + _sq_row(tnf) - 2.0 * _dot3_t(tnb, tnf), 1e-12)
        wp = jnp.exp(-d2)
        wc = 0.5 * (d_ref[pl.ds(blk, R2), :].astype(F32)
                    + e_ref[pl.ds(blk, R2), :].astype(F32))
        w = 0.5 * (wp + wc)
        rows = blk + jax.lax.broadcasted_iota(jnp.int32, (R2, N), 0)
        cols = jax.lax.broadcasted_iota(jnp.int32, (R2, N), 1)
        offd = (rows != cols).astype(F32)

        def terms(sb_, sfull):
            s = _dist_tile(sb_, sfull)
            rs = jnp.sum(s, axis=1, keepdims=True)
            sn = s * (N / rs)
            lse = jnp.log(jnp.sum(jnp.exp(-sn), axis=1, keepdims=True))
            hh = jnp.maximum(1.0 - sn, 0.0)
            loss = jnp.sum((sn * sn * w + hh * hh * (1.0 - w)) * offd,
                           keepdims=True)
            return loss, sn, lse

        lf, sfn, lsea = terms(f_ref[pl.ds(blk, R2), :], f_ref[...])
        lg, sgn, lseb = terms(g_ref[pl.ds(blk, R2), :], g_ref[...])
        p = jnp.exp(-sgn - lseb)
        kl = (jnp.sum(p * (sfn - sgn), keepdims=True)
              + jnp.sum(lsea - lseb, keepdims=True))

        @pl.when(k == 0)
        def _init():
            rcf_ref[...] = jnp.zeros((1, 1), F32)
            rcg_ref[...] = jnp.zeros((1, 1), F32)
            kl_ref[...] = jnp.zeros((1, 1), F32)

        rcf_ref[...] += lf
        rcg_ref[...] += lg
        kl_ref[...] += kl


def _00(i):
    return (0, 0)


def kernel(s_f, s_g, t_g, idx):
    idx = idx.astype(jnp.int32)
    idxc = idx.reshape(N, 1)
    idxr = idx.reshape(1, N)

    ef = pl.BlockSpec((N, D), _00)
    one = pl.BlockSpec((1, 1), _00)
    rcf, rcg, kl = pl.pallas_call(
        _mega,
        grid=(24,),
        in_specs=[ef, pl.BlockSpec((N, 1), _00), pl.BlockSpec((1, N), _00),
                  ef, ef],
        out_specs=[one, one, one],
        out_shape=[jax.ShapeDtypeStruct((1, 1), F32)] * 3,
        scratch_shapes=[pltpu.VMEM((N, N), BF16)] * 5,
        compiler_params=pltpu.CompilerParams(vmem_limit_bytes=58 * 2**20),
    )(t_g, idxc, idxr, s_f, s_g)

    scale = 1.0 / (N * (N - 1))
    loss_rc = 0.5 * (rcf[0, 0] + rcg[0, 0]) * scale
    loss_kl = kl[0, 0] / N
    return (loss_rc, loss_kl, loss_rc + loss_kl)


# speedup vs baseline: 7.5740x; 1.0323x over previous
import jax
import jax.numpy as jnp
from jax.experimental import pallas as pl
from jax.experimental.pallas import tpu as pltpu

N = 2048
D = 64
R1 = 512
R2 = 256
TK = 10
TH = 5
_DNT = (((1,), (1,)), ((), ()))
_DN = (((1,), (0,)), ((), ()))
F32 = jnp.float32
BF16 = jnp.bfloat16


def _split(x):
    hi = x.astype(BF16)
    lo = (x - hi.astype(F32)).astype(BF16)
    return hi, lo


def _dot3_t(a, b):
    ahi, alo = _split(a)
    bhi, blo = _split(b)
    out = jax.lax.dot_general(ahi, bhi, _DNT, preferred_element_type=F32)
    out = out + jax.lax.dot_general(ahi, blo, _DNT, preferred_element_type=F32)
    out = out + jax.lax.dot_general(alo, bhi, _DNT, preferred_element_type=F32)
    return out


def _sq_row(x):
    xt = jnp.transpose(x)
    return jnp.sum(xt * xt, axis=0, keepdims=True)


def _sq_col(x):
    return jnp.sum(x * x, axis=1, keepdims=True)


def _dist_tile(xb, xf):
    d2 = _sq_col(xb) + _sq_row(xf) - 2.0 * _dot3_t(xb, xf)
    return jnp.sqrt(jnp.maximum(d2, 1e-12))


def _norm_rows(x):
    n = jnp.sqrt(jnp.sum(x * x, axis=1, keepdims=True))
    return x / jnp.maximum(n, 1e-12)


def _mega(t_ref, ic_ref, ir_ref, f_ref, g_ref,
          rcf_ref, rcg_ref, kl_ref,
          a_ref, b_ref, h_ref, d_ref, e_ref):
    i = pl.program_id(0)

    @pl.when(i < 4)
    def _s2():
        blk = i * R1
        tnb = _norm_rows(t_ref[pl.ds(blk, R1), :])
        tnf = _norm_rows(t_ref[...])
        d2 = jnp.maximum(
            _sq_col(tnb) + _sq_row(tnf) - 2.0 * _dot3_t(tnb, tnf), 1e-12)
        wp = jnp.exp(-d2)
        same = ic_ref[pl.ds(blk, R1), :] == ir_ref[...]
        wpc = jnp.where(same, 1.0, wp)
        iota = jax.lax.broadcasted_iota(jnp.int32, (R1, N), 1)
        h5 = jnp.zeros((R1, N), jnp.bool_)
        for it in range(TK):
            m = jnp.max(wpc, axis=1, keepdims=True)
            j = jnp.min(jnp.where(wpc == m, iota, N), axis=1, keepdims=True)
            wpc = jnp.where(iota == j, -1.0, wpc)
            if it == TH - 1:
                h5 = wpc < 0.0
        accf = (wpc < 0.0).astype(F32)
        a_ref[pl.ds(blk, R1), :] = accf.astype(BF16)
        b_ref[:, pl.ds(blk, R1)] = jnp.transpose(accf).astype(BF16)
        h_ref[pl.ds(blk, R1), :] = h5.astype(BF16)

    @pl.when(jnp.logical_and(i >= 4, i < 8))
    def _s3a():
        blk = (i - 4) * R1
        a_ref[pl.ds(blk, R1), :] = (a_ref[pl.ds(blk, R1), :]
                                    * b_ref[pl.ds(blk, R1), :])

    @pl.when(jnp.logical_and(i >= 8, i < 12))
    def _s3b():
        blk = (i - 8) * R1
        vb = a_ref[pl.ds(blk, R1), :]
        vv = jax.lax.dot_general(vb, a_ref[...], _DNT,
                                 preferred_element_type=F32)
        rc = jnp.sum(vb.astype(F32), axis=1, keepdims=True)
        wct = vb.astype(F32) * vv / jnp.maximum(rc, 1.0)
        b_ref[pl.ds(blk, R1), :] = wct.astype(BF16)

    @pl.when(jnp.logical_and(i >= 12, i < 16))
    def _sa():
        blk = (i - 12) * R1
        w = jax.lax.dot_general(h_ref[pl.ds(blk, R1), :], b_ref[...], _DN,
                                preferred_element_type=F32) * (1.0 / TH)
        d_ref[pl.ds(blk, R1), :] = w.astype(BF16)
        e_ref[:, pl.ds(blk, R1)] = jnp.transpose(w).astype(BF16)

    @pl.when(i >= 16)
    def _sb():
        k = i - 16
        blk = k * R2
        tnb = _norm_rows(t_ref[pl.ds(blk, R2), :])
        tnf = _norm_rows(t_ref[...])
        d2 = jnp.maximum(
            _sq_col(tnb) + _sq_row(tnf) - 2.0 * _dot3_t(tnb, tnf), 1e-12)
        wp = jnp.exp(-d2)
        wc = 0.5 * (d_ref[pl.ds(blk, R2), :].astype(F32)
                    + e_ref[pl.ds(blk, R2), :].astype(F32))
        w = 0.5 * (wp + wc)
        rows = blk + jax.lax.broadcasted_iota(jnp.int32, (R2, N), 0)
        cols = jax.lax.broadcasted_iota(jnp.int32, (R2, N), 1)
        offd = (rows != cols).astype(F32)
        wo = w * offd
        wo2 = offd - wo

        def terms(sb_, sfull):
            s = _dist_tile(sb_, sfull)
            rs = jnp.sum(s, axis=1, keepdims=True)
            sn = s * (N / rs)
            es = jnp.exp(-sn)
            se = jnp.sum(es, axis=1, keepdims=True)
            lse = jnp.log(se)
            hh = jnp.maximum(1.0 - sn, 0.0)
            loss = jnp.sum(sn * sn * wo + hh * hh * wo2, keepdims=True)
            return loss, sn, lse, es, se

        lf, sfn, lsea, _, _ = terms(f_ref[pl.ds(blk, R2), :], f_ref[...])
        lg, sgn, lseb, eg, seg = terms(g_ref[pl.ds(blk, R2), :], g_ref[...])
        p = eg * (1.0 / seg)
        kl = (jnp.sum(p * (sfn - sgn), keepdims=True)
              + jnp.sum(lsea - lseb, keepdims=True))

        @pl.when(k == 0)
        def _init():
            rcf_ref[...] = jnp.zeros((1, 1), F32)
            rcg_ref[...] = jnp.zeros((1, 1), F32)
            kl_ref[...] = jnp.zeros((1, 1), F32)

        rcf_ref[...] += lf
        rcg_ref[...] += lg
        kl_ref[...] += kl


def _00(i):
    return (0, 0)


def kernel(s_f, s_g, t_g, idx):
    idx = idx.astype(jnp.int32)
    idxc = idx.reshape(N, 1)
    idxr = idx.reshape(1, N)

    ef = pl.BlockSpec((N, D), _00)
    one = pl.BlockSpec((1, 1), _00)
    rcf, rcg, kl = pl.pallas_call(
        _mega,
        grid=(24,),
        in_specs=[ef, pl.BlockSpec((N, 1), _00), pl.BlockSpec((1, N), _00),
                  ef, ef],
        out_specs=[one, one, one],
        out_shape=[jax.ShapeDtypeStruct((1, 1), F32)] * 3,
        scratch_shapes=[pltpu.VMEM((N, N), BF16)] * 5,
        compiler_params=pltpu.CompilerParams(vmem_limit_bytes=58 * 2**20),
    )(t_g, idxc, idxr, s_f, s_g)

    scale = 1.0 / (N * (N - 1))
    loss_rc = 0.5 * (rcf[0, 0] + rcg[0, 0]) * scale
    loss_kl = kl[0, 0] / N
    return (loss_rc, loss_kl, loss_rc + loss_kl)


# speedup vs baseline: 7.9548x; 1.0503x over previous
import jax
import jax.numpy as jnp
from jax.experimental import pallas as pl
from jax.experimental.pallas import tpu as pltpu

N = 2048
D = 64
R1 = 512
R2 = 256
TK = 10
TH = 5
_DNT = (((1,), (1,)), ((), ()))
_DN = (((1,), (0,)), ((), ()))
F32 = jnp.float32
BF16 = jnp.bfloat16


def _split(x):
    hi = x.astype(BF16)
    lo = (x - hi.astype(F32)).astype(BF16)
    return hi, lo


def _dot3_t(a, b):
    ahi, alo = _split(a)
    bhi, blo = _split(b)
    out = jax.lax.dot_general(ahi, bhi, _DNT, preferred_element_type=F32)
    out = out + jax.lax.dot_general(ahi, blo, _DNT, preferred_element_type=F32)
    out = out + jax.lax.dot_general(alo, bhi, _DNT, preferred_element_type=F32)
    return out


def _sq_row(x):
    xt = jnp.transpose(x)
    return jnp.sum(xt * xt, axis=0, keepdims=True)


def _sq_col(x):
    return jnp.sum(x * x, axis=1, keepdims=True)


def _dist_tile(xb, xf):
    d2 = _sq_col(xb) + _sq_row(xf) - 2.0 * _dot3_t(xb, xf)
    return jnp.sqrt(jnp.maximum(d2, 1e-12))


def _norm_rows(x):
    n = jnp.sqrt(jnp.sum(x * x, axis=1, keepdims=True))
    return x / jnp.maximum(n, 1e-12)


def _mega(t_ref, ic_ref, ir_ref, f_ref, g_ref,
          rcf_ref, rcg_ref, kl_ref,
          a_ref, b_ref, h_ref, d_ref, e_ref):
    i = pl.program_id(0)

    @pl.when(i < 4)
    def _s2():
        blk = i * R1
        tnb = _norm_rows(t_ref[pl.ds(blk, R1), :])
        tnf = _norm_rows(t_ref[...])
        d2 = jnp.maximum(
            _sq_col(tnb) + _sq_row(tnf) - 2.0 * _dot3_t(tnb, tnf), 1e-12)
        wp = jnp.exp(-d2)
        same = ic_ref[pl.ds(blk, R1), :] == ir_ref[...]
        wpc = jnp.where(same, 1.0, wp)
        iota = jax.lax.broadcasted_iota(jnp.int32, (R1, N), 1)
        h5 = jnp.zeros((R1, N), jnp.bool_)
        for it in range(TK):
            j = jnp.argmax(wpc, axis=1, keepdims=True).astype(jnp.int32)
            wpc = jnp.where(iota == j, -1.0, wpc)
            if it == TH - 1:
                h5 = wpc < 0.0
        accf = (wpc < 0.0).astype(F32)
        a_ref[pl.ds(blk, R1), :] = accf.astype(BF16)
        b_ref[:, pl.ds(blk, R1)] = jnp.transpose(accf).astype(BF16)
        h_ref[pl.ds(blk, R1), :] = h5.astype(BF16)

    @pl.when(jnp.logical_and(i >= 4, i < 8))
    def _s3a():
        blk = (i - 4) * R1
        a_ref[pl.ds(blk, R1), :] = (a_ref[pl.ds(blk, R1), :]
                                    * b_ref[pl.ds(blk, R1), :])

    @pl.when(jnp.logical_and(i >= 8, i < 12))
    def _s3b():
        blk = (i - 8) * R1
        vb = a_ref[pl.ds(blk, R1), :]
        vv = jax.lax.dot_general(vb, a_ref[...], _DNT,
                                 preferred_element_type=F32)
        rc = jnp.sum(vb.astype(F32), axis=1, keepdims=True)
        wct = vb.astype(F32) * vv / jnp.maximum(rc, 1.0)
        b_ref[pl.ds(blk, R1), :] = wct.astype(BF16)

    @pl.when(jnp.logical_and(i >= 12, i < 16))
    def _sa():
        blk = (i - 12) * R1
        w = jax.lax.dot_general(h_ref[pl.ds(blk, R1), :], b_ref[...], _DN,
                                preferred_element_type=F32) * (1.0 / TH)
        d_ref[pl.ds(blk, R1), :] = w.astype(BF16)
        e_ref[:, pl.ds(blk, R1)] = jnp.transpose(w).astype(BF16)

    @pl.when(i >= 16)
    def _sb():
        k = i - 16
        blk = k * R2
        tnb = _norm_rows(t_ref[pl.ds(blk, R2), :])
        tnf = _norm_rows(t_ref[...])
        d2 = jnp.maximum(
            _sq_col(tnb) + _sq_row(tnf) - 2.0 * _dot3_t(tnb, tnf), 1e-12)
        wp = jnp.exp(-d2)
        wc = 0.5 * (d_ref[pl.ds(blk, R2), :].astype(F32)
                    + e_ref[pl.ds(blk, R2), :].astype(F32))
        w = 0.5 * (wp + wc)
        rows = blk + jax.lax.broadcasted_iota(jnp.int32, (R2, N), 0)
        cols = jax.lax.broadcasted_iota(jnp.int32, (R2, N), 1)
        offd = (rows != cols).astype(F32)
        wo = w * offd
        wo2 = offd - wo

        def terms(sb_, sfull):
            s = _dist_tile(sb_, sfull)
            rs = jnp.sum(s, axis=1, keepdims=True)
            sn = s * (N / rs)
            es = jnp.exp(-sn)
            se = jnp.sum(es, axis=1, keepdims=True)
            lse = jnp.log(se)
            hh = jnp.maximum(1.0 - sn, 0.0)
            loss = jnp.sum(sn * sn * wo + hh * hh * wo2, keepdims=True)
            return loss, sn, lse, es, se

        lf, sfn, lsea, _, _ = terms(f_ref[pl.ds(blk, R2), :], f_ref[...])
        lg, sgn, lseb, eg, seg = terms(g_ref[pl.ds(blk, R2), :], g_ref[...])
        p = eg * (1.0 / seg)
        kl = (jnp.sum(p * (sfn - sgn), keepdims=True)
              + jnp.sum(lsea - lseb, keepdims=True))

        @pl.when(k == 0)
        def _init():
            rcf_ref[...] = jnp.zeros((1, 1), F32)
            rcg_ref[...] = jnp.zeros((1, 1), F32)
            kl_ref[...] = jnp.zeros((1, 1), F32)

        rcf_ref[...] += lf
        rcg_ref[...] += lg
        kl_ref[...] += kl


def _00(i):
    return (0, 0)


def kernel(s_f, s_g, t_g, idx):
    idx = idx.astype(jnp.int32)
    idxc = idx.reshape(N, 1)
    idxr = idx.reshape(1, N)

    ef = pl.BlockSpec((N, D), _00)
    one = pl.BlockSpec((1, 1), _00)
    rcf, rcg, kl = pl.pallas_call(
        _mega,
        grid=(24,),
        in_specs=[ef, pl.BlockSpec((N, 1), _00), pl.BlockSpec((1, N), _00),
                  ef, ef],
        out_specs=[one, one, one],
        out_shape=[jax.ShapeDtypeStruct((1, 1), F32)] * 3,
        scratch_shapes=[pltpu.VMEM((N, N), BF16)] * 5,
        compiler_params=pltpu.CompilerParams(vmem_limit_bytes=58 * 2**20),
    )(t_g, idxc, idxr, s_f, s_g)

    scale = 1.0 / (N * (N - 1))
    loss_rc = 0.5 * (rcf[0, 0] + rcg[0, 0]) * scale
    loss_kl = kl[0, 0] / N
    return (loss_rc, loss_kl, loss_rc + loss_kl)
